# baseline (speedup 1.0000x reference)
import jax
import jax.numpy as jnp
from jax.experimental import pallas as pl
from jax.experimental.pallas import tpu as pltpu

_INV_SQRT2 = 0.7071067811865476


def _gelu(x):
    return 0.5 * x * (1.0 + jax.lax.erf(x * _INV_SQRT2))


def _dot_t(x, w):
    return jax.lax.dot_general(
        x, w, dimension_numbers=(((1,), (1,)), ((), ())),
        preferred_element_type=jnp.float32)


def _make_body(Bb, V, H, I):
    def _body(idx_ref, vs_ref, vals_ref, W1_ref, b1_ref, W2_ref, b2_ref,
              G1_ref, g1_ref, G2_ref, g2_ref, out_ref):
        vs = vs_ref[...]
        idx = idx_ref[...]
        colbs = [jax.lax.broadcast_in_dim(idx[:, i], (Bb, H), (0,))
                 for i in range(I)]

        origs = []
        valss = []
        for i in range(I):
            colb = colbs[i]
            acc = jnp.where(colb == 0, vs[:, 0:H], 0.0)
            for v in range(1, V):
                acc = jnp.where(colb == v, vs[:, v * H:(v + 1) * H], acc)
            origs.append(acc)
            valss.append(vals_ref[:, i * H:(i + 1) * H])
        orig = jnp.concatenate(origs, axis=0)
        vals = jnp.concatenate(valss, axis=0)

        W1a = W1_ref[:, :H]
        W1b = W1_ref[:, H:]
        h = _gelu(_dot_t(orig, W1a) + _dot_t(vals, W1b) + b1_ref[...])
        enc = _dot_t(h, W2_ref[...]) + b2_ref[...]
        g = _gelu(_dot_t(enc, G1_ref[...]) + g1_ref[...])
        gate = jax.nn.sigmoid(
            jnp.sum(g * G2_ref[...], axis=-1, keepdims=True) + g2_ref[0, 0])
        newv = orig * (1.0 - gate) + vals * gate

        for v in range(V):
            row = vs[:, v * H:(v + 1) * H]
            for i in range(I):
                row = jnp.where(colbs[i] == v,
                                newv[i * Bb:(i + 1) * Bb, :], row)
            out_ref[:, v * H:(v + 1) * H] = row
    return _body


@jax.jit
def _run(variable_states, intervention_indices, intervention_values,
         W1, b1, W2, b2, G1, g1, G2, g2):
    B, V, H = variable_states.shape
    I = intervention_indices.shape[1]
    Bb = 64
    grid = (B // Bb,)
    vs2 = variable_states.reshape(B, V * H)
    vals2 = intervention_values.reshape(B, I * H)
    b1r = b1.reshape(1, H)
    b2r = b2.reshape(1, H)
    g1r = g1.reshape(1, H)
    g2r = g2.reshape(1, 1)
    full = lambda *shape: pl.BlockSpec(shape, lambda b: (0,) * len(shape))
    out = pl.pallas_call(
        _make_body(Bb, V, H, I),
        grid=grid,
        in_specs=[
            pl.BlockSpec((Bb, I), lambda b: (b, 0)),
            pl.BlockSpec((Bb, V * H), lambda b: (b, 0)),
            pl.BlockSpec((Bb, I * H), lambda b: (b, 0)),
            full(H, 2 * H),
            full(1, H),
            full(H, H),
            full(1, H),
            full(H, H),
            full(1, H),
            full(1, H),
            full(1, 1),
        ],
        out_specs=pl.BlockSpec((Bb, V * H), lambda b: (b, 0)),
        out_shape=jax.ShapeDtypeStruct((B, V * H), jnp.float32),
        compiler_params=pltpu.CompilerParams(
            dimension_semantics=("arbitrary",)),
    )(intervention_indices, vs2, vals2,
      W1, b1r, W2, b2r, G1, g1r, G2, g2r)
    return out.reshape(B, V, H)


def kernel(variable_states, edge_probs, intervention_indices,
           intervention_values, W1, b1, W2, b2, G1, g1, G2, g2):
    del edge_probs
    return _run(variable_states, intervention_indices, intervention_values,
                W1, b1, W2, b2, G1, g1, G2, g2)


# speedup vs baseline: 1.9478x; 1.9478x over previous
import jax
import jax.numpy as jnp
from jax.experimental import pallas as pl
from jax.experimental.pallas import tpu as pltpu

_INV_SQRT2 = 0.7071067811865476


def _gelu(x):
    return 0.5 * x * (1.0 + jax.lax.erf(x * _INV_SQRT2))


def _dot_t(x, w):
    return jax.lax.dot_general(
        x, w, dimension_numbers=(((1,), (1,)), ((), ())),
        preferred_element_type=jnp.float32)


def _make_body(Bb, V, H, I):
    def _body(idx_ref, vs_ref, vals_ref, W1_ref, b1_ref, W2_ref, b2_ref,
              G1_ref, g1_ref, G2_ref, g2_ref, out_ref):
        vs = vs_ref[...]
        idx = idx_ref[...]

        idx_f = idx.astype(jnp.float32)
        sup_cols = []
        for i in range(I):
            s = None
            for j in range(i + 1, I):
                c = (idx[:, i:i + 1] == idx[:, j:j + 1])
                s = c if s is None else jnp.logical_or(s, c)
            if s is None:
                sup_cols.append(jnp.zeros((Bb, 1), jnp.float32))
            else:
                sup_cols.append(jnp.where(s, 1.0, 0.0))
        combo = jnp.concatenate([idx_f] + sup_cols, axis=1)

        ii = jax.lax.broadcasted_iota(jnp.int32, (Bb, Bb), 0)
        jj = jax.lax.broadcasted_iota(jnp.int32, (Bb, Bb), 1)
        eye = jnp.where(ii == jj, 1.0, 0.0)
        comboT = jax.lax.dot_general(
            combo, eye, dimension_numbers=(((0,), (0,)), ((), ())),
            preferred_element_type=jnp.float32)

        lane_b = jax.lax.broadcasted_iota(jnp.int32, (1, Bb), 1)
        siota = jax.lax.broadcasted_iota(jnp.int32, (Bb * V, Bb), 0)
        chunks = []
        for i in range(I):
            tt = lane_b * V + comboT[i:i + 1, :].astype(jnp.int32)
            tt = jnp.where(comboT[I + i:I + i + 1, :] > 0.5, -1, tt)
            ttb = jax.lax.broadcast_in_dim(tt, (Bb * V, Bb), (0, 1))
            chunks.append(jnp.where(siota == ttb, 1.0, 0.0))
        S = jnp.concatenate(chunks, axis=1)

        orig = jax.lax.dot_general(
            S, vs, dimension_numbers=(((0,), (0,)), ((), ())),
            preferred_element_type=jnp.float32)
        vals = jnp.concatenate(
            [vals_ref[:, i * H:(i + 1) * H] for i in range(I)], axis=0)

        W1a = W1_ref[:, :H]
        W1b = W1_ref[:, H:]
        h = _gelu(_dot_t(orig, W1a) + _dot_t(vals, W1b) + b1_ref[...])
        enc = _dot_t(h, W2_ref[...]) + b2_ref[...]
        g = _gelu(_dot_t(enc, G1_ref[...]) + g1_ref[...])
        gate = jax.nn.sigmoid(
            jnp.sum(g * G2_ref[...], axis=-1, keepdims=True) + g2_ref[0, 0])
        delta = gate * (vals - orig)

        out_ref[...] = vs + jax.lax.dot_general(
            S, delta, dimension_numbers=(((1,), (0,)), ((), ())),
            preferred_element_type=jnp.float32)
    return _body


@jax.jit
def _run(variable_states, intervention_indices, intervention_values,
         W1, b1, W2, b2, G1, g1, G2, g2):
    B, V, H = variable_states.shape
    I = intervention_indices.shape[1]
    Bb = 64
    grid = (B // Bb,)
    vs2 = variable_states.reshape(B * V, H)
    vals2 = intervention_values.reshape(B, I * H)
    b1r = b1.reshape(1, H)
    b2r = b2.reshape(1, H)
    g1r = g1.reshape(1, H)
    g2r = g2.reshape(1, 1)
    full = lambda *shape: pl.BlockSpec(shape, lambda b: (0,) * len(shape))
    out = pl.pallas_call(
        _make_body(Bb, V, H, I),
        grid=grid,
        in_specs=[
            pl.BlockSpec((Bb, I), lambda b: (b, 0)),
            pl.BlockSpec((Bb * V, H), lambda b: (b, 0)),
            pl.BlockSpec((Bb, I * H), lambda b: (b, 0)),
            full(H, 2 * H),
            full(1, H),
            full(H, H),
            full(1, H),
            full(H, H),
            full(1, H),
            full(1, H),
            full(1, 1),
        ],
        out_specs=pl.BlockSpec((Bb * V, H), lambda b: (b, 0)),
        out_shape=jax.ShapeDtypeStruct((B * V, H), jnp.float32),
        compiler_params=pltpu.CompilerParams(
            dimension_semantics=("arbitrary",)),
    )(intervention_indices, vs2, vals2,
      W1, b1r, W2, b2r, G1, g1r, G2, g2r)
    return out.reshape(B, V, H)


def kernel(variable_states, edge_probs, intervention_indices,
           intervention_values, W1, b1, W2, b2, G1, g1, G2, g2):
    del edge_probs
    return _run(variable_states, intervention_indices, intervention_values,
                W1, b1, W2, b2, G1, g1, G2, g2)
